# baseline (speedup 1.0000x reference)
import functools

import jax
import jax.numpy as jnp
from jax import lax
from jax.experimental import pallas as pl
from jax.experimental.pallas import tpu as pltpu

_VMEM_LIMIT = 64 * 1024 * 1024


def _qkv_kernel(x_ref, wq_ref, wk_ref, wv_ref, q_ref, k_ref, v_ref,
                *, n_groups, rep, head_dim):
    G, R, D = n_groups, rep, head_dim
    xb = x_ref[...].astype(jnp.bfloat16)
    qt = jnp.dot(xb, wq_ref[...],
                 preferred_element_type=jnp.float32).astype(jnp.bfloat16)
    kt = jnp.dot(xb, wk_ref[...],
                 preferred_element_type=jnp.float32).astype(jnp.bfloat16)
    vt = jnp.dot(xb, wv_ref[...],
                 preferred_element_type=jnp.float32).astype(jnp.bfloat16)
    for g in range(G):
        for r in range(R):
            h = g * R + r
            q_ref[0, g, r, :, :] = qt[:, h * D:(h + 1) * D]
        k_ref[0, g, :, :] = kt[:, g * D:(g + 1) * D]
        v_ref[0, g, :, :] = vt[:, g * D:(g + 1) * D]


def _qkv_proj(x2, wq, wk, wv, *, B, T, n_groups, rep, head_dim, tile_m=128):
    M, K = x2.shape
    Nq = wq.shape[1]
    Nk = wk.shape[1]
    G, R, D = n_groups, rep, head_dim
    tpb = T // tile_m
    grid = (M // tile_m,)
    body = functools.partial(_qkv_kernel, n_groups=G, rep=R, head_dim=D)
    return pl.pallas_call(
        body,
        out_shape=(jax.ShapeDtypeStruct((B, G, R, T, D), jnp.bfloat16),
                   jax.ShapeDtypeStruct((B, G, T, D), jnp.bfloat16),
                   jax.ShapeDtypeStruct((B, G, T, D), jnp.bfloat16)),
        grid=grid,
        in_specs=[
            pl.BlockSpec((tile_m, K), lambda i: (i, 0)),
            pl.BlockSpec((K, Nq), lambda i: (0, 0)),
            pl.BlockSpec((K, Nk), lambda i: (0, 0)),
            pl.BlockSpec((K, Nk), lambda i: (0, 0)),
        ],
        out_specs=(
            pl.BlockSpec((1, G, R, tile_m, D),
                         lambda i: (i // tpb, 0, 0, i % tpb, 0)),
            pl.BlockSpec((1, G, tile_m, D),
                         lambda i: (i // tpb, 0, i % tpb, 0)),
            pl.BlockSpec((1, G, tile_m, D),
                         lambda i: (i // tpb, 0, i % tpb, 0)),
        ),
        compiler_params=pltpu.CompilerParams(
            dimension_semantics=("parallel",),
            vmem_limit_bytes=_VMEM_LIMIT,
        ),
    )(x2, wq, wk, wv)


def _attn_oproj_kernel(q_ref, k_ref, v_ref, wo_ref, out_ref,
                       stats_ref, acc_ref,
                       *, n_groups, rep, head_dim, tq, tk):
    G, R, D = n_groups, rep, head_dim
    S = R * tq
    qi = pl.program_id(1)
    ki = pl.program_id(2)

    allowed = (lax.broadcasted_iota(jnp.int32, (S, tk), 1) <=
               jnp.bitwise_and(lax.broadcasted_iota(jnp.int32, (S, tk), 0),
                               tq - 1))

    def _update(masked, first):
        for g in range(G):
            q_s = q_ref[0, g].reshape(S, D)
            k_g = k_ref[0, g, pl.ds(ki * tk, tk), :]
            v_g = v_ref[0, g, pl.ds(ki * tk, tk), :]
            s = lax.dot_general(q_s, k_g, (((1,), (1,)), ((), ())),
                                preferred_element_type=jnp.float32)
            if masked:
                s = jnp.where(allowed, s, -1e30)
            if first:
                m_new = jnp.max(s, axis=-1, keepdims=True)
                p = jnp.exp(s - m_new)
                stats_ref[g, :, 1:2] = jnp.sum(p, axis=-1, keepdims=True)
                acc_ref[g] = jnp.dot(p.astype(v_g.dtype), v_g,
                                     preferred_element_type=jnp.float32)
            else:
                m_prev = stats_ref[g, :, 0:1]
                m_new = jnp.maximum(m_prev,
                                    jnp.max(s, axis=-1, keepdims=True))
                alpha = jnp.exp(m_prev - m_new)
                p = jnp.exp(s - m_new)
                stats_ref[g, :, 1:2] = (alpha * stats_ref[g, :, 1:2]
                                        + jnp.sum(p, axis=-1, keepdims=True))
                pv = jnp.dot(p.astype(v_g.dtype), v_g,
                             preferred_element_type=jnp.float32)
                acc_ref[g] = alpha * acc_ref[g] + pv
            stats_ref[g, :, 0:1] = m_new

    first = ki == 0
    diag = ki == qi
    needed = ki <= qi

    @pl.when(jnp.logical_and(first, diag))
    def _first_diag():
        _update(masked=True, first=True)

    @pl.when(jnp.logical_and(first, jnp.logical_not(diag)))
    def _first_fast():
        _update(masked=False, first=True)

    @pl.when(jnp.logical_and(jnp.logical_not(first), diag))
    def _rest_diag():
        _update(masked=True, first=False)

    @pl.when(jnp.logical_and(
        jnp.logical_not(first),
        jnp.logical_and(jnp.logical_not(diag), needed)))
    def _rest_fast():
        _update(masked=False, first=False)

    @pl.when(ki == pl.num_programs(2) - 1)
    def _finalize():
        cols = []
        for g in range(G):
            inv_l = pl.reciprocal(
                jnp.maximum(stats_ref[g, :, 1:2], 1e-30), approx=True)
            og = (acc_ref[g] * inv_l).astype(jnp.bfloat16)
            for r in range(R):
                cols.append(og[r * tq:(r + 1) * tq, :])
        o_tile = jnp.concatenate(cols, axis=1)
        out_ref[0] = jnp.dot(o_tile, wo_ref[...],
                             preferred_element_type=jnp.float32)


def _flash_attn_oproj(q, k, v, wo, *, n_groups, rep, head_dim,
                      tq=256, tk=256):
    B, G, R, T, D = q.shape
    HD = G * R * D
    dim = wo.shape[1]
    grid = (B, T // tq, T // tk)

    body = functools.partial(_attn_oproj_kernel, n_groups=G, rep=R,
                             head_dim=D, tq=tq, tk=tk)
    return pl.pallas_call(
        body,
        out_shape=jax.ShapeDtypeStruct((B, T, dim), jnp.float32),
        grid=grid,
        in_specs=[
            pl.BlockSpec((1, G, R, tq, D), lambda b, qi, ki: (b, 0, 0, qi, 0)),
            pl.BlockSpec((1, G, T, D), lambda b, qi, ki: (b, 0, 0, 0)),
            pl.BlockSpec((1, G, T, D), lambda b, qi, ki: (b, 0, 0, 0)),
            pl.BlockSpec((HD, dim), lambda b, qi, ki: (0, 0)),
        ],
        out_specs=pl.BlockSpec((1, tq, dim), lambda b, qi, ki: (b, qi, 0)),
        scratch_shapes=[
            pltpu.VMEM((G, R * tq, 128), jnp.float32),
            pltpu.VMEM((G, R * tq, D), jnp.float32),
        ],
        compiler_params=pltpu.CompilerParams(
            dimension_semantics=("parallel", "parallel", "arbitrary"),
            vmem_limit_bytes=_VMEM_LIMIT,
        ),
    )(q, k, v, wo)


def kernel(x, wq_c, wk_c, wv_c, wo_c):
    B, T, dim = x.shape
    HD = wq_c.shape[1]
    GD = wk_c.shape[1]
    D = 128
    H = HD // D
    G = GD // D
    R = H // G

    x2 = x.reshape(B * T, dim)
    q, k, v = _qkv_proj(x2, wq_c, wk_c, wv_c,
                        B=B, T=T, n_groups=G, rep=R, head_dim=D)
    out = _flash_attn_oproj(q, k, v, wo_c,
                            n_groups=G, rep=R, head_dim=D)
    return out


# speedup vs baseline: 1.0446x; 1.0446x over previous
import functools

import jax
import jax.numpy as jnp
from jax import lax
from jax.experimental import pallas as pl
from jax.experimental.pallas import tpu as pltpu

_VMEM_LIMIT = 64 * 1024 * 1024


def _qkv_kernel(x_ref, wq_ref, wk_ref, wv_ref, q_ref, k_ref, v_ref,
                *, n_groups, rep, head_dim):
    G, R, D = n_groups, rep, head_dim
    xb = x_ref[...].astype(jnp.bfloat16)
    qt = jnp.dot(xb, wq_ref[...],
                 preferred_element_type=jnp.float32).astype(jnp.bfloat16)
    kt = jnp.dot(xb, wk_ref[...],
                 preferred_element_type=jnp.float32).astype(jnp.bfloat16)
    vt = jnp.dot(xb, wv_ref[...],
                 preferred_element_type=jnp.float32).astype(jnp.bfloat16)
    for g in range(G):
        for r in range(R):
            h = g * R + r
            q_ref[0, g, r, :, :] = qt[:, h * D:(h + 1) * D]
        k_ref[0, g, :, :] = kt[:, g * D:(g + 1) * D]
        v_ref[0, g, :, :] = vt[:, g * D:(g + 1) * D]


def _qkv_proj(x2, wq, wk, wv, *, B, T, n_groups, rep, head_dim, tile_m=128):
    M, K = x2.shape
    Nq = wq.shape[1]
    Nk = wk.shape[1]
    G, R, D = n_groups, rep, head_dim
    tpb = T // tile_m
    grid = (M // tile_m,)
    body = functools.partial(_qkv_kernel, n_groups=G, rep=R, head_dim=D)
    return pl.pallas_call(
        body,
        out_shape=(jax.ShapeDtypeStruct((B, G, R, T, D), jnp.bfloat16),
                   jax.ShapeDtypeStruct((B, G, T, D), jnp.bfloat16),
                   jax.ShapeDtypeStruct((B, G, T, D), jnp.bfloat16)),
        grid=grid,
        in_specs=[
            pl.BlockSpec((tile_m, K), lambda i: (i, 0)),
            pl.BlockSpec((K, Nq), lambda i: (0, 0)),
            pl.BlockSpec((K, Nk), lambda i: (0, 0)),
            pl.BlockSpec((K, Nk), lambda i: (0, 0)),
        ],
        out_specs=(
            pl.BlockSpec((1, G, R, tile_m, D),
                         lambda i: (i // tpb, 0, 0, i % tpb, 0)),
            pl.BlockSpec((1, G, tile_m, D),
                         lambda i: (i // tpb, 0, i % tpb, 0)),
            pl.BlockSpec((1, G, tile_m, D),
                         lambda i: (i // tpb, 0, i % tpb, 0)),
        ),
        compiler_params=pltpu.CompilerParams(
            dimension_semantics=("parallel",),
            vmem_limit_bytes=_VMEM_LIMIT,
        ),
    )(x2, wq, wk, wv)


def _attn_oproj_kernel(q_ref, k_ref, v_ref, wo_ref, out_ref,
                       m_ref, l_ref, acc_ref,
                       *, n_groups, rep, head_dim, tq, tk):
    G, R, D = n_groups, rep, head_dim
    S = R * tq
    qi = pl.program_id(1)
    ki = pl.program_id(2)

    allowed = (lax.broadcasted_iota(jnp.int32, (S, tk), 1) <=
               jnp.bitwise_and(lax.broadcasted_iota(jnp.int32, (S, tk), 0),
                               tq - 1))

    def _update(masked, first):
        for g in range(G):
            q_s = q_ref[0, g].reshape(S, D)
            k_g = k_ref[0, g]
            v_g = v_ref[0, g]
            s = lax.dot_general(q_s, k_g, (((1,), (1,)), ((), ())),
                                preferred_element_type=jnp.float32)
            if masked:
                s = jnp.where(allowed, s, -1e30)
            if first:
                m_new = jnp.max(s, axis=-1, keepdims=True)
                p = jnp.exp(s - m_new)
                l_ref[g] = jnp.sum(p, axis=-1, keepdims=True)
                acc_ref[g] = jnp.dot(p.astype(v_g.dtype), v_g,
                                     preferred_element_type=jnp.float32)
            else:
                m_prev = m_ref[g]
                m_new = jnp.maximum(m_prev,
                                    jnp.max(s, axis=-1, keepdims=True))
                alpha = jnp.exp(m_prev - m_new)
                p = jnp.exp(s - m_new)
                l_ref[g] = alpha * l_ref[g] + jnp.sum(p, axis=-1,
                                                      keepdims=True)
                pv = jnp.dot(p.astype(v_g.dtype), v_g,
                             preferred_element_type=jnp.float32)
                acc_ref[g] = alpha * acc_ref[g] + pv
            m_ref[g] = m_new

    first = ki == 0
    diag = ki == qi
    needed = ki <= qi

    @pl.when(jnp.logical_and(first, diag))
    def _first_diag():
        _update(masked=True, first=True)

    @pl.when(jnp.logical_and(first, jnp.logical_not(diag)))
    def _first_fast():
        _update(masked=False, first=True)

    @pl.when(jnp.logical_and(jnp.logical_not(first), diag))
    def _rest_diag():
        _update(masked=True, first=False)

    @pl.when(jnp.logical_and(
        jnp.logical_not(first),
        jnp.logical_and(jnp.logical_not(diag), needed)))
    def _rest_fast():
        _update(masked=False, first=False)

    @pl.when(ki == pl.num_programs(2) - 1)
    def _finalize():
        cols = []
        for g in range(G):
            inv_l = pl.reciprocal(jnp.maximum(l_ref[g], 1e-30), approx=True)
            og = (acc_ref[g] * inv_l).astype(jnp.bfloat16)
            for r in range(R):
                cols.append(og[r * tq:(r + 1) * tq, :])
        o_tile = jnp.concatenate(cols, axis=1)
        out_ref[0] = jnp.dot(o_tile, wo_ref[...],
                             preferred_element_type=jnp.float32)


def _flash_attn_oproj(q, k, v, wo, *, n_groups, rep, head_dim,
                      tq=256, tk=256):
    B, G, R, T, D = q.shape
    HD = G * R * D
    dim = wo.shape[1]
    grid = (B, T // tq, T // tk)

    def kv_map(b, qi, ki):
        last = (qi * tq + (tq - 1)) // tk
        return (b, 0, jnp.minimum(ki, last), 0)

    body = functools.partial(_attn_oproj_kernel, n_groups=G, rep=R,
                             head_dim=D, tq=tq, tk=tk)
    return pl.pallas_call(
        body,
        out_shape=jax.ShapeDtypeStruct((B, T, dim), jnp.float32),
        grid=grid,
        in_specs=[
            pl.BlockSpec((1, G, R, tq, D), lambda b, qi, ki: (b, 0, 0, qi, 0)),
            pl.BlockSpec((1, G, tk, D), kv_map),
            pl.BlockSpec((1, G, tk, D), kv_map),
            pl.BlockSpec((HD, dim), lambda b, qi, ki: (0, 0)),
        ],
        out_specs=pl.BlockSpec((1, tq, dim), lambda b, qi, ki: (b, qi, 0)),
        scratch_shapes=[
            pltpu.VMEM((G, R * tq, 1), jnp.float32),
            pltpu.VMEM((G, R * tq, 1), jnp.float32),
            pltpu.VMEM((G, R * tq, D), jnp.float32),
        ],
        compiler_params=pltpu.CompilerParams(
            dimension_semantics=("parallel", "parallel", "arbitrary"),
            vmem_limit_bytes=_VMEM_LIMIT,
        ),
    )(q, k, v, wo)


def kernel(x, wq_c, wk_c, wv_c, wo_c):
    B, T, dim = x.shape
    HD = wq_c.shape[1]
    GD = wk_c.shape[1]
    D = 128
    H = HD // D
    G = GD // D
    R = H // G

    x2 = x.reshape(B * T, dim)
    q, k, v = _qkv_proj(x2, wq_c, wk_c, wv_c,
                        B=B, T=T, n_groups=G, rep=R, head_dim=D)
    out = _flash_attn_oproj(q, k, v, wo_c,
                            n_groups=G, rep=R, head_dim=D)
    return out
